# baseline (speedup 1.0000x reference)
import functools

import jax
import jax.numpy as jnp
import numpy as np
from jax import lax
from jax.experimental import pallas as pl
from jax.experimental.pallas import tpu as pltpu
from jax.experimental.pallas import tpu_sc as plsc

_INPUT_DIM = 4096
_NUM_EXPERTS = 64
_TOP_K = 8
_NOISE_EPS = 0.01
_BATCH = 8192
_BLK = 512

_NC = 2
_NS = 16
_NW = _NC * _NS
_RPW = _BATCH // _NW
_GRP = 16
_NGRP = _RPW // _GRP
_ILP = 2

_STD_NORMAL = np.asarray(
    jax.random.normal(jax.random.key(42), (_BATCH, _NUM_EXPERTS),
                      dtype=jnp.float32))


def _gating_kernel(x_ref, w_ref, std_ref, bias_ref, gk_ref, idx3_ref,
                   load_ref):
    logits = jnp.dot(x_ref[...], w_ref[...],
                     preferred_element_type=jnp.float32)
    clean = logits[:, :_NUM_EXPERTS]
    raw_noise = logits[:, _NUM_EXPERTS:]
    noise = std_ref[...] * jax.nn.softplus(raw_noise) * _NOISE_EPS
    gates = jax.nn.sigmoid(clean + noise)
    bias_gates = gates + bias_ref[...]

    iota = lax.broadcasted_iota(jnp.int32, (_BLK, _NUM_EXPERTS), 1)
    work = bias_gates
    gk_cols = []
    idx_cols = []
    part = jnp.zeros((1, _NUM_EXPERTS), jnp.float32)
    for _ in range(_TOP_K):
        m = jnp.max(work, axis=1, keepdims=True)
        cand = jnp.where(work == m, iota, _NUM_EXPERTS)
        idx = jnp.min(cand, axis=1, keepdims=True)
        sel = iota == idx
        gk_cols.append(jnp.sum(jnp.where(sel, gates, 0.0), axis=1,
                               keepdims=True))
        idx_cols.append(idx)
        part = part + jnp.sum(sel.astype(jnp.float32), axis=0,
                              keepdims=True)
        work = jnp.where(sel, -jnp.inf, work)

    gk_ref[...] = jnp.concatenate(gk_cols, axis=1)
    idx_mat = jnp.concatenate(idx_cols, axis=1).astype(jnp.float32)
    for h in range(_BLK // _RPW):
        idx3_ref[h] = idx_mat[h * _RPW:(h + 1) * _RPW, :].T

    @pl.when(pl.program_id(0) == 0)
    def _init():
        load_ref[...] = jnp.zeros_like(load_ref)

    load_ref[...] += part * (1.0 / (_BATCH * _TOP_K))


def _gates_tc(x, w_comb, std, bias_row):
    return pl.pallas_call(
        _gating_kernel,
        grid=(_BATCH // _BLK,),
        in_specs=[
            pl.BlockSpec((_BLK, _INPUT_DIM), lambda i: (i, 0)),
            pl.BlockSpec((_INPUT_DIM, 2 * _NUM_EXPERTS), lambda i: (0, 0)),
            pl.BlockSpec((_BLK, _NUM_EXPERTS), lambda i: (i, 0)),
            pl.BlockSpec((1, _NUM_EXPERTS), lambda i: (0, 0)),
        ],
        out_specs=[
            pl.BlockSpec((_BLK, _TOP_K), lambda i: (i, 0)),
            pl.BlockSpec((_BLK // _RPW, _TOP_K, _RPW), lambda i: (i, 0, 0)),
            pl.BlockSpec((1, _NUM_EXPERTS), lambda i: (0, 0)),
        ],
        out_shape=[
            jax.ShapeDtypeStruct((_BATCH, _TOP_K), jnp.float32),
            jax.ShapeDtypeStruct((_NW, _TOP_K, _RPW), jnp.float32),
            jax.ShapeDtypeStruct((1, _NUM_EXPERTS), jnp.float32),
        ],
        compiler_params=pltpu.CompilerParams(
            dimension_semantics=("arbitrary",)),
    )(x, w_comb, std, bias_row)


_SC_MESH = plsc.VectorSubcoreMesh(core_axis_name="c", subcore_axis_name="s")

_GATHER_DN = lax.GatherDimensionNumbers(
    offset_dims=(), collapsed_slice_dims=(0,), start_index_map=(0,))


def _vgather(vec, idx):
    return lax.gather(vec, idx.reshape(16, 1), _GATHER_DN, (1,),
                      mode=lax.GatherScatterMode.PROMISE_IN_BOUNDS)


@functools.partial(
    pl.kernel,
    mesh=_SC_MESH,
    out_type=[
        jax.ShapeDtypeStruct((_BATCH // (_ILP * _GRP), _ILP * _GRP,
                              _NUM_EXPERTS), jnp.int32),
    ],
    scratch_types=[
        pltpu.VMEM((_TOP_K, _RPW), jnp.float32),
        pltpu.VMEM((_ILP * _GRP, _NUM_EXPERTS), jnp.int32),
    ],
)
def _scatter_sc(idx3_hbm, r3_hbm, slab, rg):
    wid = lax.axis_index("s") * _NC + lax.axis_index("c")
    pltpu.sync_copy(idx3_hbm.at[wid], slab)

    lane = lax.iota(jnp.int32, 16)
    zeros_i = jnp.zeros((16,), jnp.int32)

    def group_body(g, carry):
        col0 = g * (_GRP * _ILP)
        for h in range(_ILP):
            lo = zeros_i
            hi = zeros_i
            for j in range(_TOP_K):
                e = slab[j, pl.ds(col0 + h * _GRP, _GRP)].astype(jnp.int32)
                bit = jnp.int32(1) << (e & 15)
                bit16 = jnp.where((e & 16) != 0, bit << 16, bit)
                lo = lo | jnp.where(e < 32, bit16, 0)
                hi = hi | jnp.where(e >= 32, bit16, 0)

            for r in range(_GRP):
                rsplat = jnp.full((16,), r, jnp.int32)
                lo_r = _vgather(lo, rsplat)
                hi_r = _vgather(hi, rsplat)
                for c in range(4):
                    src = lo_r if c < 2 else hi_r
                    onehot = (src >> (lane + (c % 2) * 16)) & 1
                    rg[h * _GRP + r, pl.ds(c * 16, 16)] = onehot

        gid = wid * (_NGRP // _ILP) + g
        pltpu.sync_copy(rg, r3_hbm.at[gid])
        return carry

    lax.fori_loop(0, _NGRP // _ILP, group_body, 0)


def kernel(x, w_gate, w_noise, bias):
    w_comb = jnp.concatenate([w_gate, w_noise], axis=0).T
    std = jnp.asarray(_STD_NORMAL)
    gk, idx3, load = _gates_tc(x, w_comb, std,
                               bias.reshape(1, _NUM_EXPERTS))
    (r3,) = _scatter_sc(idx3)
    router = r3.reshape(_BATCH, _NUM_EXPERTS)
    return gk, router, load.reshape(_NUM_EXPERTS)
